# baseline (speedup 1.0000x reference)
import jax
import jax.numpy as jnp
from jax import lax
from jax.experimental import pallas as pl
from jax.experimental.pallas import tpu as pltpu
from jax.experimental.pallas import tpu_sc as plsc

_CAP = 50
_MEM = 128
_B = 4096
_ROWS = _B * _CAP

_NC = 2
_NS = 16
_NW = _NC * _NS
_BPW = _B // _NW
_L = 16


def _copy_body(src, dst):
    dst[...] = src[...]


_COPY_RB = 6400
_tc_copy = pl.pallas_call(
    _copy_body,
    grid=(_ROWS // _COPY_RB,),
    in_specs=[pl.BlockSpec((_COPY_RB, _MEM), lambda i: (i, 0))],
    out_specs=pl.BlockSpec((_COPY_RB, _MEM), lambda i: (i, 0)),
    out_shape=jax.ShapeDtypeStruct((_ROWS, _MEM), jnp.float32),
)


def _sc_body(inputs_hbm, cnt_hbm, out_ref, cnt_out_hbm,
             cnt_v, idx_v, cnt1_v, rows_v, sem):
    wid = lax.axis_index("s") * _NC + lax.axis_index("c")
    base = wid * _BPW
    pltpu.sync_copy(cnt_hbm.at[pl.ds(base, _BPW)], cnt_v)
    for i in range(_BPW // _L):
        cv = cnt_v[pl.ds(i * _L, _L)]
        slot = lax.rem(cv, _CAP)
        brow = (base + i * _L) + lax.iota(jnp.int32, _L)
        idx_v[pl.ds(i * _L, _L)] = brow * _CAP + slot
        cnt1_v[pl.ds(i * _L, _L)] = cv + 1
    pltpu.sync_copy(cnt1_v, cnt_out_hbm.at[pl.ds(base, _BPW)])
    pltpu.sync_copy(inputs_hbm.at[pl.ds(base, _BPW)], rows_v)
    pltpu.async_copy(rows_v, out_ref.at[idx_v], sem).wait()


_sc_scatter = pl.kernel(
    _sc_body,
    out_type=jax.ShapeDtypeStruct((_B,), jnp.int32),
    mesh=plsc.VectorSubcoreMesh(core_axis_name="c", subcore_axis_name="s",
                                num_cores=_NC, num_subcores=_NS),
    scratch_types=[
        pltpu.VMEM((_BPW,), jnp.int32),
        pltpu.VMEM((_BPW,), jnp.int32),
        pltpu.VMEM((_BPW,), jnp.int32),
        pltpu.VMEM((_BPW, _MEM), jnp.float32),
        pltpu.SemaphoreType.DMA,
    ],
)


def kernel(inputs, cnt, mem):
    cnt = cnt.astype(jnp.int32)
    mem2d = mem.reshape(_ROWS, _MEM)
    out_ref = jax.new_ref(_tc_copy(mem2d))
    counter = _sc_scatter(inputs, cnt, out_ref)
    memories = jax.freeze(out_ref).reshape(_B, _CAP, _MEM)
    return (memories, counter, memories)
